# Initial kernel scaffold; baseline (speedup 1.0000x reference)
import jax
import jax.numpy as jnp
from jax.experimental import pallas as pl


def kernel(x, special_tokens, pretrained):
    raise NotImplementedError("write your pallas kernel here")


# speedup vs baseline: 1.0065x; 1.0065x over previous
import functools

import jax
import jax.numpy as jnp
from jax import lax
from jax.experimental import pallas as pl
from jax.experimental.pallas import tpu as pltpu
from jax.experimental.pallas import tpu_sc as plsc

_D = 64
_B = 16384 * 50
_NC = 2
_NS = 16
_NW = _NC * _NS
_BPW = _B // _NW
_G = 128
_NG = _BPW // _G
_SCALE = 8.0


def _emb_body(x_hbm, spec_hbm, pretr_hbm, out_hbm, idx_v, adj_v, rows_v, spec_v, bad_v, gsem):
    wid = lax.axis_index("s") * _NC + lax.axis_index("c")
    base = wid * _BPW
    pltpu.sync_copy(x_hbm.at[pl.ds(base, _BPW)], idx_v)
    pltpu.sync_copy(spec_hbm, spec_v)

    for r in range(5):
        for cc in range(_D // 16):
            spec_v[r, pl.ds(cc * 16, 16)] = spec_v[r, pl.ds(cc * 16, 16)] * _SCALE

    bad_v[pl.ds(0, 16)] = jnp.zeros((16,), jnp.int32)

    def per_group(g, c):
        off = g * _G
        bad = bad_v[pl.ds(0, 16)]
        for j in range(_G // 16):
            v = idx_v[pl.ds(off + j * 16, 16)]
            adj_v[pl.ds(j * 16, 16)] = jnp.maximum(v - 5, 0)
            bad = bad | (v < 5).astype(jnp.int32)
        bad_v[pl.ds(0, 16)] = bad
        pltpu.async_copy(pretr_hbm.at[adj_v], rows_v, gsem).wait()

        def scale_row(r, cc_):
            for cc in range(_D // 16):
                rows_v[r, pl.ds(cc * 16, 16)] = rows_v[r, pl.ds(cc * 16, 16)] * _SCALE
            return cc_

        lax.fori_loop(0, _G, scale_row, 0)
        pltpu.sync_copy(rows_v, out_hbm.at[pl.ds(base + off, _G)])
        return c

    lax.fori_loop(0, _NG, per_group, 0)

    @pl.when(jnp.max(bad_v[pl.ds(0, 16)]) > 0)
    def _fixup():
        def scan_vreg(i, c):
            v = idx_v[pl.ds(i * 16, 16)]

            @pl.when(jnp.any(v < 5))
            def _():
                for lane in range(16):
                    t = v[lane]

                    @pl.when(t < 5)
                    def _():
                        pltpu.sync_copy(
                            spec_v.at[pl.ds(t, 1)],
                            out_hbm.at[pl.ds(base + i * 16 + lane, 1)],
                        )

            return c

        lax.fori_loop(0, _BPW // 16, scan_vreg, 0)


@jax.jit
def _emb_lookup(x_flat, special_tokens, pretrained):
    run = functools.partial(
        pl.kernel,
        out_type=jax.ShapeDtypeStruct((_B, _D), jnp.float32),
        mesh=plsc.VectorSubcoreMesh(core_axis_name="c", subcore_axis_name="s"),
        compiler_params=pltpu.CompilerParams(
            use_tc_tiling_on_sc=False, needs_layout_passes=False
        ),
        scratch_types=[
            pltpu.VMEM((_BPW,), jnp.int32),
            pltpu.VMEM((_G,), jnp.int32),
            pltpu.VMEM((_G, _D), jnp.float32),
            pltpu.VMEM((5, _D), jnp.float32),
            pltpu.VMEM((16,), jnp.int32),
            pltpu.SemaphoreType.DMA,
        ],
    )(_emb_body)
    return run(x_flat, special_tokens, pretrained)


def kernel(x, special_tokens, pretrained):
    x_flat = x.reshape(-1).astype(jnp.int32)
    out = _emb_lookup(x_flat, special_tokens, pretrained)
    return out.reshape(x.shape + (_D,))


# speedup vs baseline: 1.2003x; 1.1926x over previous
import functools

import jax
import jax.numpy as jnp
from jax import lax
from jax.experimental import pallas as pl
from jax.experimental.pallas import tpu as pltpu
from jax.experimental.pallas import tpu_sc as plsc

_D = 64
_B = 16384 * 50
_NC = 2
_NS = 16
_NW = _NC * _NS
_BPW = _B // _NW
_G = 128
_NG = _BPW // _G
_NBUF = 4
_SCALE = 8.0


def _emb_body(x_hbm, spec_hbm, pretr_hbm, out_hbm, idx_v, adj_v, rows_v, spec_v, bad_v, gsem, osem):
    wid = lax.axis_index("s") * _NC + lax.axis_index("c")
    base = wid * _BPW
    pltpu.sync_copy(x_hbm.at[pl.ds(base, _BPW)], idx_v)
    pltpu.sync_copy(spec_hbm, spec_v)

    for r in range(5):
        for cc in range(_D // 16):
            spec_v[r, pl.ds(cc * 16, 16)] = spec_v[r, pl.ds(cc * 16, 16)] * _SCALE

    bad_v[pl.ds(0, 16)] = jnp.zeros((16,), jnp.int32)

    def adj_of(g, slot):
        off = g * _G
        bad = bad_v[pl.ds(0, 16)]
        for j in range(_G // 16):
            v = idx_v[pl.ds(off + j * 16, 16)]
            adj_v[slot, pl.ds(j * 16, 16)] = jnp.maximum(v - 5, 0)
            bad = bad | (v < 5).astype(jnp.int32)
        bad_v[pl.ds(0, 16)] = bad

    def fire_gather(slot):
        pltpu.async_copy(pretr_hbm.at[adj_v.at[slot]], rows_v.at[slot], gsem)

    def wait_gather(slot):
        pltpu.make_async_copy(
            pretr_hbm.at[adj_v.at[slot]], rows_v.at[slot], gsem
        ).wait()

    def fire_scatter(g, slot):
        pltpu.async_copy(rows_v.at[slot], out_hbm.at[pl.ds(base + g * _G, _G)], osem)

    def wait_scatter(g, slot):
        pltpu.make_async_copy(
            rows_v.at[slot], out_hbm.at[pl.ds(base + g * _G, _G)], osem
        ).wait()

    adj_of(0, 0)
    fire_gather(0)
    adj_of(1, 1)
    fire_gather(1)

    def outer(go, c):
        for b in range(_NBUF):
            g = go * _NBUF + b
            wait_gather(b)
            nslot = (b + 2) % _NBUF

            @pl.when(g + 2 < _NG)
            def _():
                @pl.when(g >= 2)
                def _():
                    wait_scatter(g - 2, nslot)

                adj_of(g + 2, nslot)
                fire_gather(nslot)

            def scale4(r4, cc_):
                for rr in range(4):
                    for cc in range(_D // 16):
                        rows_v[b, r4 * 4 + rr, pl.ds(cc * 16, 16)] = (
                            rows_v[b, r4 * 4 + rr, pl.ds(cc * 16, 16)] * _SCALE
                        )
                return cc_

            lax.fori_loop(0, _G // 4, scale4, 0)
            fire_scatter(g, b)
        return c

    lax.fori_loop(0, _NG // _NBUF, outer, 0)
    for k in range(_NBUF):
        g = _NG - _NBUF + k
        wait_scatter(g, g % _NBUF)

    @pl.when(jnp.max(bad_v[pl.ds(0, 16)]) > 0)
    def _fixup():
        def scan_vreg(i, c):
            v = idx_v[pl.ds(i * 16, 16)]

            @pl.when(jnp.any(v < 5))
            def _():
                for lane in range(16):
                    t = v[lane]

                    @pl.when(t < 5)
                    def _():
                        pltpu.sync_copy(
                            spec_v.at[pl.ds(t, 1)],
                            out_hbm.at[pl.ds(base + i * 16 + lane, 1)],
                        )

            return c

        lax.fori_loop(0, _BPW // 16, scan_vreg, 0)


@jax.jit
def _emb_lookup(x_flat, special_tokens, pretrained):
    run = functools.partial(
        pl.kernel,
        out_type=jax.ShapeDtypeStruct((_B, _D), jnp.float32),
        mesh=plsc.VectorSubcoreMesh(core_axis_name="c", subcore_axis_name="s"),
        compiler_params=pltpu.CompilerParams(
            use_tc_tiling_on_sc=False, needs_layout_passes=False
        ),
        scratch_types=[
            pltpu.VMEM((_BPW,), jnp.int32),
            pltpu.VMEM((_NBUF, _G), jnp.int32),
            pltpu.VMEM((_NBUF, _G, _D), jnp.float32),
            pltpu.VMEM((5, _D), jnp.float32),
            pltpu.VMEM((16,), jnp.int32),
            pltpu.SemaphoreType.DMA,
            pltpu.SemaphoreType.DMA,
        ],
    )(_emb_body)
    return run(x_flat, special_tokens, pretrained)


def kernel(x, special_tokens, pretrained):
    x_flat = x.reshape(-1).astype(jnp.int32)
    out = _emb_lookup(x_flat, special_tokens, pretrained)
    return out.reshape(x.shape + (_D,))
